# Initial kernel scaffold; baseline (speedup 1.0000x reference)
import jax
import jax.numpy as jnp
from jax.experimental import pallas as pl


def kernel(scores, order):
    raise NotImplementedError("write your pallas kernel here")


# speedup vs baseline: 76.5796x; 76.5796x over previous
import functools

import jax
import jax.numpy as jnp
from jax import lax
from jax.experimental import pallas as pl
from jax.experimental.pallas import tpu as pltpu
from jax.experimental.pallas import tpu_sc as plsc

_LANES = 16


@functools.cache
def _make_sc_scatter(B, N):
    info = plsc.get_sparse_core_info()
    NC, NS = info.num_cores, info.num_subcores
    NW = NC * NS
    assert B % NW == 0
    RPW = B // NW
    CH = -(-N // _LANES)
    ROWW = RPW * N
    mesh = plsc.VectorSubcoreMesh(core_axis_name="c", subcore_axis_name="s")

    @functools.partial(
        pl.kernel,
        out_type=jax.ShapeDtypeStruct((B * N,), jnp.float32),
        mesh=mesh,
        compiler_params=pltpu.CompilerParams(needs_layout_passes=False),
        scratch_types=[
            pltpu.VMEM((ROWW + _LANES,), jnp.float32),
            pltpu.VMEM((ROWW + _LANES,), jnp.int32),
            pltpu.VMEM((ROWW,), jnp.float32),
        ],
    )
    def sc_scatter(scores_hbm, order_hbm, out_hbm, sc_v, od_v, out_v):
        wid = lax.axis_index("s") * NC + lax.axis_index("c")
        base = wid * ROWW
        pltpu.sync_copy(scores_hbm.at[pl.ds(base, ROWW)], sc_v.at[pl.ds(0, ROWW)])
        pltpu.sync_copy(order_hbm.at[pl.ds(base, ROWW)], od_v.at[pl.ds(0, ROWW)])
        lane = lax.iota(jnp.int32, _LANES)

        def row_body(r, carry):
            rowoff = r * N
            for ci in range(CH):
                off = rowoff + ci * _LANES
                s = sc_v[pl.ds(off, _LANES)]
                o = od_v[pl.ds(off, _LANES)]
                idx = o - 1 + rowoff
                if (ci + 1) * _LANES <= N:
                    plsc.store_scatter(out_v, [idx], s)
                else:
                    plsc.store_scatter(out_v, [idx], s,
                                       mask=lane < (N - ci * _LANES))
            return carry

        lax.fori_loop(0, RPW, row_body, 0)
        pltpu.sync_copy(out_v, out_hbm.at[pl.ds(base, ROWW)])

    return sc_scatter


@functools.cache
def _make_tc_loss(B, N):
    def body(sorted_ref, out_ref):
        s = sorted_ref[...]
        m = jnp.max(s, axis=1, keepdims=True)
        e = jnp.exp(s - m)
        kk = lax.broadcasted_iota(jnp.int32, (N, N), 0)
        jj = lax.broadcasted_iota(jnp.int32, (N, N), 1)
        tri = (kk >= jj).astype(jnp.float32)
        ssum = lax.dot_general(e, tri, (((1,), (0,)), ((), ())),
                               precision=lax.Precision.HIGHEST,
                               preferred_element_type=jnp.float32)
        lse = m + jnp.log(ssum)
        loss = (jnp.sum(lse) - jnp.sum(s)) / (B * N)
        out_ref[...] = loss[None, None]

    return pl.pallas_call(
        body,
        out_shape=jax.ShapeDtypeStruct((1, 1), jnp.float32),
    )


def kernel(scores, order):
    B, N = scores.shape
    sorted_flat = _make_sc_scatter(B, N)(scores.reshape(-1),
                                         order.reshape(-1).astype(jnp.int32))
    loss = _make_tc_loss(B, N)(sorted_flat.reshape(B, N))
    return loss[0, 0]


# speedup vs baseline: 90.3498x; 1.1798x over previous
import functools

import jax
import jax.numpy as jnp
from jax import lax
from jax.experimental import pallas as pl
from jax.experimental.pallas import tpu as pltpu
from jax.experimental.pallas import tpu_sc as plsc

_LANES = 16
_STRIDE = 256


@functools.cache
def _make_sc_scatter(B, N):
    info = plsc.get_sparse_core_info()
    NC, NS = info.num_cores, info.num_subcores
    NW = NC * NS
    assert B % NW == 0 and N % 8 == 0 and N < _STRIDE
    RPW = B // NW
    OUTW = RPW * _STRIDE
    offs = list(range(0, N - _LANES, _LANES)) + [N - _LANES]
    mesh = plsc.VectorSubcoreMesh(core_axis_name="c", subcore_axis_name="s")

    HR = RPW // 2
    HOUT = HR * _STRIDE

    @functools.partial(
        pl.kernel,
        out_type=jax.ShapeDtypeStruct((B * _STRIDE,), jnp.float32),
        mesh=mesh,
        compiler_params=pltpu.CompilerParams(needs_layout_passes=False,
                                             use_tc_tiling_on_sc=True),
        scratch_types=[
            pltpu.VMEM((RPW, N), jnp.float32),
            pltpu.VMEM((RPW, N), jnp.int32),
            pltpu.VMEM((OUTW,), jnp.float32),
            pltpu.SemaphoreType.DMA,
            pltpu.SemaphoreType.DMA,
            pltpu.SemaphoreType.DMA,
        ],
    )
    def sc_scatter(scores_hbm, order_hbm, out_hbm, sc_v, od_v, out_v,
                   sem_a, sem_b, sem_o):
        wid = lax.axis_index("s") * NC + lax.axis_index("c")
        base = wid * RPW
        ca1 = pltpu.async_copy(scores_hbm.at[pl.ds(base, HR)],
                               sc_v.at[pl.ds(0, HR)], sem_a)
        ca2 = pltpu.async_copy(order_hbm.at[pl.ds(base, HR)],
                               od_v.at[pl.ds(0, HR)], sem_a)
        cb1 = pltpu.async_copy(scores_hbm.at[pl.ds(base + HR, HR)],
                               sc_v.at[pl.ds(HR, HR)], sem_b)
        cb2 = pltpu.async_copy(order_hbm.at[pl.ds(base + HR, HR)],
                               od_v.at[pl.ds(HR, HR)], sem_b)

        def row_body(r, carry):
            out_off = r * _STRIDE
            for co in offs:
                s = sc_v[r, pl.ds(co, _LANES)]
                o = od_v[r, pl.ds(co, _LANES)]
                plsc.store_scatter(out_v, [o - 1 + out_off], s)
            return carry

        ca1.wait()
        ca2.wait()
        lax.fori_loop(0, HR, row_body, 0, unroll=4)
        co1 = pltpu.async_copy(out_v.at[pl.ds(0, HOUT)],
                               out_hbm.at[pl.ds(wid * OUTW, HOUT)], sem_o)
        cb1.wait()
        cb2.wait()
        lax.fori_loop(HR, RPW, row_body, 0, unroll=4)
        co1.wait()
        pltpu.sync_copy(out_v.at[pl.ds(HOUT, HOUT)],
                        out_hbm.at[pl.ds(wid * OUTW + HOUT, HOUT)])

    return sc_scatter


@functools.cache
def _make_tc_loss(B, N):
    VR = 2 * B
    TAIL = N - 128
    BLK = VR
    G = VR // BLK

    def body(a_ref, out_ref):
        neg = jnp.full((1, 1), -jnp.inf, jnp.float32)
        zero = jnp.zeros((1, 1), jnp.float32)
        even = lax.broadcasted_iota(jnp.int32, (BLK, 1), 0) % 2 == 0
        lane = lax.broadcasted_iota(jnp.int32, (BLK, 128), 1)
        valid = even | (lane < TAIL)
        a = jnp.where(valid, a_ref[...], -jnp.inf)
        mrow = jnp.max(a, axis=1, keepdims=True)
        up = jnp.concatenate([mrow[1:], neg], axis=0)
        down = jnp.concatenate([neg, mrow[:-1]], axis=0)
        m = jnp.maximum(mrow, jnp.where(even, up, down))
        e = jnp.exp(a - m)
        kk = lax.broadcasted_iota(jnp.int32, (128, 128), 0)
        jj = lax.broadcasted_iota(jnp.int32, (128, 128), 1)
        tri = (kk >= jj).astype(jnp.float32)
        s = lax.dot_general(e, tri, (((1,), (0,)), ((), ())),
                            precision=lax.Precision.HIGHEST,
                            preferred_element_type=jnp.float32)
        tot_next = jnp.concatenate([s[1:, 0:1], zero], axis=0)
        s = s + jnp.where(even, tot_next, 0.0)
        lse_sum = jnp.sum(jnp.where(valid, jnp.log(s) + m, 0.0))
        score_sum = jnp.sum(jnp.where(valid, a, 0.0))
        part = (lse_sum - score_sum) / (B * N)

        @pl.when(pl.program_id(0) == 0)
        def _():
            out_ref[...] = jnp.zeros((1, 1), jnp.float32)

        out_ref[...] += part[None, None]

    return pl.pallas_call(
        body,
        grid=(G,),
        in_specs=[pl.BlockSpec((BLK, 128), lambda i: (i, 0))],
        out_specs=pl.BlockSpec((1, 1), lambda i: (0, 0)),
        out_shape=jax.ShapeDtypeStruct((1, 1), jnp.float32),
    )


def kernel(scores, order):
    B, N = scores.shape
    sorted_flat = _make_sc_scatter(B, N)(scores, order.astype(jnp.int32))
    loss = _make_tc_loss(B, N)(sorted_flat.reshape(2 * B, 128))
    return loss[0, 0]


# speedup vs baseline: 91.2562x; 1.0100x over previous
import functools

import jax
import jax.numpy as jnp
from jax import lax
from jax.experimental import pallas as pl
from jax.experimental.pallas import tpu as pltpu
from jax.experimental.pallas import tpu_sc as plsc

_LANES = 16
_STRIDE = 256


@functools.cache
def _make_sc_scatter(B, N):
    info = plsc.get_sparse_core_info()
    NC, NS = info.num_cores, info.num_subcores
    NW = NC * NS
    assert B % NW == 0 and N % 8 == 0 and N < _STRIDE
    RPW = B // NW
    OUTW = RPW * _STRIDE
    offs = list(range(0, N - _LANES, _LANES)) + [N - _LANES]
    mesh = plsc.VectorSubcoreMesh(core_axis_name="c", subcore_axis_name="s")

    HR = RPW // 2
    HOUT = HR * _STRIDE

    @functools.partial(
        pl.kernel,
        out_type=jax.ShapeDtypeStruct((B * _STRIDE,), jnp.float32),
        mesh=mesh,
        compiler_params=pltpu.CompilerParams(needs_layout_passes=False,
                                             use_tc_tiling_on_sc=True),
        scratch_types=[
            pltpu.VMEM((RPW, N), jnp.float32),
            pltpu.VMEM((RPW, N), jnp.int32),
            pltpu.VMEM((OUTW,), jnp.float32),
            pltpu.SemaphoreType.DMA,
            pltpu.SemaphoreType.DMA,
            pltpu.SemaphoreType.DMA,
        ],
    )
    def sc_scatter(scores_hbm, order_hbm, out_hbm, sc_v, od_v, out_v,
                   sem_a, sem_b, sem_o):
        wid = lax.axis_index("s") * NC + lax.axis_index("c")
        base = wid * RPW
        ca1 = pltpu.async_copy(scores_hbm.at[pl.ds(base, HR)],
                               sc_v.at[pl.ds(0, HR)], sem_a)
        ca2 = pltpu.async_copy(order_hbm.at[pl.ds(base, HR)],
                               od_v.at[pl.ds(0, HR)], sem_a)
        cb1 = pltpu.async_copy(scores_hbm.at[pl.ds(base + HR, HR)],
                               sc_v.at[pl.ds(HR, HR)], sem_b)
        cb2 = pltpu.async_copy(order_hbm.at[pl.ds(base + HR, HR)],
                               od_v.at[pl.ds(HR, HR)], sem_b)

        def row_body(r, carry):
            out_off = r * _STRIDE
            for co in offs:
                s = sc_v[r, pl.ds(co, _LANES)]
                o = od_v[r, pl.ds(co, _LANES)]
                plsc.store_scatter(out_v, [o - 1 + out_off], s)
            return carry

        ca1.wait()
        ca2.wait()
        lax.fori_loop(0, HR, row_body, 0)
        co1 = pltpu.async_copy(out_v.at[pl.ds(0, HOUT)],
                               out_hbm.at[pl.ds(wid * OUTW, HOUT)], sem_o)
        cb1.wait()
        cb2.wait()
        lax.fori_loop(HR, RPW, row_body, 0)
        co1.wait()
        pltpu.sync_copy(out_v.at[pl.ds(HOUT, HOUT)],
                        out_hbm.at[pl.ds(wid * OUTW + HOUT, HOUT)])

    return sc_scatter


@functools.cache
def _make_tc_loss(B, N):
    VR = 2 * B
    TAIL = N - 128
    BLK = VR
    G = VR // BLK

    def body(a_ref, out_ref):
        neg = jnp.full((1, 1), -jnp.inf, jnp.float32)
        zero = jnp.zeros((1, 1), jnp.float32)
        even = lax.broadcasted_iota(jnp.int32, (BLK, 1), 0) % 2 == 0
        lane = lax.broadcasted_iota(jnp.int32, (BLK, 128), 1)
        valid = even | (lane < TAIL)
        a = jnp.where(valid, a_ref[...], -jnp.inf)
        mrow = jnp.max(a, axis=1, keepdims=True)
        up = jnp.concatenate([mrow[1:], neg], axis=0)
        down = jnp.concatenate([neg, mrow[:-1]], axis=0)
        m = jnp.maximum(mrow, jnp.where(even, up, down))
        e = jnp.exp(a - m)
        kk = lax.broadcasted_iota(jnp.int32, (128, 128), 0)
        jj = lax.broadcasted_iota(jnp.int32, (128, 128), 1)
        tri = (kk >= jj).astype(jnp.float32)
        s = lax.dot_general(e, tri, (((1,), (0,)), ((), ())),
                            precision=lax.Precision.HIGHEST,
                            preferred_element_type=jnp.float32)
        tot_next = jnp.concatenate([s[1:, 0:1], zero], axis=0)
        s = s + jnp.where(even, tot_next, 0.0)
        lse_sum = jnp.sum(jnp.where(valid, jnp.log(s) + m, 0.0))
        score_sum = jnp.sum(jnp.where(valid, a, 0.0))
        part = (lse_sum - score_sum) / (B * N)

        @pl.when(pl.program_id(0) == 0)
        def _():
            out_ref[...] = jnp.zeros((1, 1), jnp.float32)

        out_ref[...] += part[None, None]

    return pl.pallas_call(
        body,
        grid=(G,),
        in_specs=[pl.BlockSpec((BLK, 128), lambda i: (i, 0))],
        out_specs=pl.BlockSpec((1, 1), lambda i: (0, 0)),
        out_shape=jax.ShapeDtypeStruct((1, 1), jnp.float32),
    )


def kernel(scores, order):
    B, N = scores.shape
    sorted_flat = _make_sc_scatter(B, N)(scores, order.astype(jnp.int32))
    loss = _make_tc_loss(B, N)(sorted_flat.reshape(2 * B, 128))
    return loss[0, 0]
